# baseline (speedup 1.0000x reference)
import jax
import jax.numpy as jnp
from jax.experimental import pallas as pl

_N = 10000
_D = 128
_LAYERS = 4


def _relu_body(x_ref, o_ref):
    o_ref[...] = jnp.maximum(x_ref[...], 0.0)


def _pallas_relu(x):
    return pl.pallas_call(
        _relu_body,
        out_shape=jax.ShapeDtypeStruct(x.shape, x.dtype),
    )(x)


def kernel(feats, edge_index, betas):
    src = edge_index[0]
    dst = edge_index[1]
    x = feats
    for i in range(_LAYERS):
        norm = jnp.linalg.norm(x, axis=-1, keepdims=True)
        xn = x / (norm + 1e-12)
        cos = jnp.sum(xn[src] * xn[dst], axis=-1)
        e = betas[i] * cos
        m = jax.ops.segment_max(e, dst, num_segments=_N)
        m = jnp.where(jnp.isfinite(m), m, 0.0)
        ex = jnp.exp(e - m[dst])
        s = jax.ops.segment_sum(ex, dst, num_segments=_N)
        alpha = ex / (s[dst] + 1e-12)
        out = jax.ops.segment_sum(alpha[:, None] * x[src], dst, num_segments=_N)
        x = _pallas_relu(out)
    return x


# speedup vs baseline: 10.8098x; 10.8098x over previous
import functools

import jax
import jax.numpy as jnp
from jax import lax
from jax.experimental import pallas as pl
from jax.experimental.pallas import tpu as pltpu
from jax.experimental.pallas import tpu_sc as plsc

_N = 10000
_E = 320000
_D = 128
_LAYERS = 4
_NC = 2
_NS = 16
_NW = _NC * _NS
_EPW = _E // _NW
_NP = 10240
_B = 80
_NBLK = _EPW // _B
_RPS = _NP // _NS


def _invn_body(x_ref, invn_ref):
    x = x_ref[...]
    ss = jnp.sum(x * x, axis=1)
    invn_ref[...] = 1.0 / (jnp.sqrt(ss) + 1e-12)


def _invn(x):
    return pl.pallas_call(
        _invn_body,
        out_shape=jax.ShapeDtypeStruct((_N,), jnp.float32),
    )(x)


def _combine_body(acc_ref, s_ref, x_ref, invn_ref):
    a = acc_ref[0, :_N] + acc_ref[1, :_N]
    s = jnp.sum(s_ref[:, :_N], axis=0)
    x = jnp.maximum(a / (s[:, None] + 1e-12), 0.0)
    x_ref[...] = x
    ss = jnp.sum(x * x, axis=1)
    invn_ref[...] = 1.0 / (jnp.sqrt(ss) + 1e-12)


def _combine(acc, s):
    return pl.pallas_call(
        _combine_body,
        out_shape=[
            jax.ShapeDtypeStruct((_N, _D), jnp.float32),
            jax.ShapeDtypeStruct((_N,), jnp.float32),
        ],
    )(acc, s)


_mesh = plsc.VectorSubcoreMesh(core_axis_name="c", subcore_axis_name="s")


@functools.partial(
    pl.kernel,
    out_type=(
        jax.ShapeDtypeStruct((_NC, _NP, _D), jnp.float32),
        jax.ShapeDtypeStruct((_NW, _NP), jnp.float32),
    ),
    mesh=_mesh,
    compiler_params=pltpu.CompilerParams(
        needs_layout_passes=False, use_tc_tiling_on_sc=False),
    scratch_types=[
        pltpu.VMEM((_B,), jnp.int32),
        pltpu.VMEM((_B,), jnp.int32),
        pltpu.VMEM((_B, _D), jnp.float32),
        pltpu.VMEM((_B, _D), jnp.float32),
        pltpu.VMEM((_N,), jnp.float32),
        pltpu.VMEM((16,), jnp.float32),
        pltpu.VMEM((256,), jnp.float32),
        pltpu.VMEM((_NP,), jnp.float32),
        pltpu.VMEM_SHARED((_NP, _D), jnp.float32),
        pltpu.SemaphoreType.DMA,
        pltpu.SemaphoreType.DMA,
    ],
)
def _edge(x_hbm, invn_hbm, src_hbm, dst_hbm, beta_hbm, zrow_hbm,
          out_hbm, sden_hbm,
          idx_s, idx_d, rows_s, rows_d, invn_v,
          beta_v, part, s_v, acc_sh, sem_s, sem_d):
    cid = lax.axis_index("c")
    sid = lax.axis_index("s")
    wid = cid * _NS + sid

    def invcp(i, c):
        pltpu.sync_copy(invn_hbm.at[pl.ds(i * 2000, 2000)],
                        invn_v.at[pl.ds(i * 2000, 2000)])
        return c

    lax.fori_loop(0, 5, invcp, 0)
    pltpu.sync_copy(beta_hbm, beta_v)

    zero16 = jnp.zeros((16,), jnp.float32)

    def zinit(i, c):
        s_v[pl.ds(i * 16, 16)] = zero16
        return c

    lax.fori_loop(0, _NP // 16, zinit, 0)

    pltpu.sync_copy(zrow_hbm, acc_sh.at[pl.ds(sid * _RPS, _RPS)])
    plsc.subcore_barrier()

    lane = lax.iota(jnp.int32, 16)
    beta = beta_v[...]

    base = wid * _EPW

    def block(b, carry):
        off = base + b * _B
        pltpu.sync_copy(src_hbm.at[pl.ds(off, _B)], idx_s)
        pltpu.sync_copy(dst_hbm.at[pl.ds(off, _B)], idx_d)
        cp_s = pltpu.async_copy(x_hbm.at[idx_s], rows_s, sem_s)
        cp_d = pltpu.async_copy(x_hbm.at[idx_d], rows_d, sem_d)
        cp_s.wait()
        cp_d.wait()
        def group(g, c2):
            gb = g * 16
            sv = idx_s[pl.ds(gb, 16)]
            dv = idx_d[pl.ds(gb, 16)]
            inv_s = plsc.load_gather(invn_v, [sv])
            inv_d = plsc.load_gather(invn_v, [dv])
            for r in range(16):
                e = gb + r
                p = rows_s[e, pl.ds(0, 16)] * rows_d[e, pl.ds(0, 16)]
                for j in range(1, 8):
                    p = p + rows_s[e, pl.ds(16 * j, 16)] * rows_d[e, pl.ds(16 * j, 16)]
                part[pl.ds(r * 16, 16)] = p
            dots = plsc.load_gather(part, [lane * 16])
            for c in range(1, 16):
                dots = dots + plsc.load_gather(part, [lane * 16 + c])
            cos = dots * inv_s * inv_d
            ex = jnp.exp(beta * cos)
            plsc.addupdate_scatter(s_v, [dv], ex)
            for r in range(16):
                e = gb + r
                coef = ex[r]
                for j in range(8):
                    rows_s[e, pl.ds(16 * j, 16)] = rows_s[e, pl.ds(16 * j, 16)] * coef
            return c2

        lax.fori_loop(0, _B // 16, group, 0)
        pltpu.sync_copy(rows_s, acc_sh.at[idx_d], add=True)
        return carry

    lax.fori_loop(0, _NBLK, block, 0)

    pltpu.sync_copy(s_v, sden_hbm.at[wid])
    plsc.subcore_barrier()
    pltpu.sync_copy(acc_sh.at[pl.ds(sid * _RPS, _RPS)],
                    out_hbm.at[cid, pl.ds(sid * _RPS, _RPS)])


def kernel(feats, edge_index, betas):
    src = edge_index[0]
    dst = edge_index[1]
    zrow = jnp.zeros((_RPS, _D), jnp.float32)
    x = feats
    invn = _invn(feats)
    for i in range(_LAYERS):
        beta_vec = jnp.full((16,), betas[i], jnp.float32)
        acc, sden = _edge(x, invn, src, dst, beta_vec, zrow)
        x, invn = _combine(acc, sden)
    return x
